# baseline (speedup 1.0000x reference)
import jax
import jax.numpy as jnp
from jax.experimental import pallas as pl
from jax.experimental.pallas import tpu as pltpu

_B, _T, _D, _R = 4, 2048, 512, 1024
_N_INNER, _N_B = 9, 2
_RATIO = 1.0 / 5.0
_NB = _N_B + _N_INNER + _N_B
_RBLK = 128
_NRB = _R // _RBLK
_NSTEPS = _B * _NRB
_THALF = _NB * _RBLK

_SELU_ALPHA = 1.6732632423543772
_SELU_SCALE = 1.0507009873554805


def _selu(x):
    return _SELU_SCALE * jnp.where(x > 0, x, _SELU_ALPHA * (jnp.exp(x) - 1.0))


def _roi_kernel(lo_ref,
                w_ref,
                feats_ref,
                wl_ref,
                wi_ref,
                wr_ref,
                bl_ref, bi_ref, br_ref,
                out_ref,
                tile_ref,
                f1_ref):
    g = pl.program_id(0)

    @pl.when(jnp.logical_and(g % _NRB == 0, g < _NSTEPS))
    def _():
        for c in range(_T // 256):
            f1_ref[pl.ds(c * 256, 256), 0, :] = feats_ref[0, pl.ds(c * 256, 256), :]

    blk = jnp.minimum(g, _NSTEPS - 1)
    blk_base = blk * _RBLK * _NB
    goff = (g % 2) * _THALF
    for mi in range(_RBLK):
        for nb in range(_NB):
            idx = blk_base + mi * _NB + nb
            lo = lo_ref[idx]
            w = w_ref[idx]
            pair = f1_ref[pl.ds(lo, 2), 0, :]
            v = pair[0:1, :] + w * (pair[1:2, :] - pair[0:1, :])
            tile_ref[pl.ds(goff + nb * _RBLK + mi, 1), 0, :] = v

    moff = ((g + 1) % 2) * _THALF

    def xv(nb):
        return tile_ref[pl.ds(moff + nb * _RBLK, _RBLK), 0, :]

    def dot(x, w):
        return jnp.dot(x, w, preferred_element_type=jnp.float32)

    acc_l = dot(xv(0), wl_ref[0])
    for j in range(1, 7):
        acc_l = acc_l + dot(xv(j), wl_ref[j])
    left = _selu(acc_l + bl_ref[0, :])

    acc_r = dot(xv(6), wl_ref[0])
    for j in range(1, 7):
        acc_r = acc_r + dot(xv(6 + j), wl_ref[j])
    right = _selu(acc_r + bl_ref[0, :])

    part1 = dot(right - left, wr_ref[0])

    acc_i = dot(xv(2), wi_ref[0])
    for j in range(1, 9):
        acc_i = acc_i + dot(xv(2 + j), wi_ref[j])
    inner = _selu(acc_i + bi_ref[0, :])

    out_ref[0, :, :] = _selu(part1 + dot(inner, wr_ref[1]) + br_ref[0, :])


def _bin_centers(a, b, n):
    i = (jnp.arange(n, dtype=a.dtype) + 0.5) / n
    return a[..., None] + i * (b - a)[..., None]


@jax.jit
def kernel(features, start_rois, end_rois, rois, rois_mask, rois_pos_emb,
           W_left, b_left, W_inner, b_inner, W_roi, b_roi):
    del start_rois, end_rois, rois_mask, rois_pos_emb

    s, e = rois[..., 0], rois[..., 1]
    ext = _RATIO * (e - s)
    pos = jnp.concatenate([
        _bin_centers(s - ext, s + ext, _N_B),
        _bin_centers(s, e, _N_INNER),
        _bin_centers(e - ext, e + ext, _N_B)], axis=-1)
    pos = jnp.clip(pos, 0.0, _T - 1)
    lo = jnp.minimum(pos.astype(jnp.int32), _T - 2)
    w = pos - lo.astype(jnp.float32)
    lo_flat = lo.reshape(-1)
    w_flat = w.reshape(-1)

    wl = W_left.reshape(_D, _D, 7).transpose(2, 1, 0)
    wi = W_inner.reshape(_D, _D, 9).transpose(2, 1, 0)
    wr = W_roi.T.reshape(2, _D, _D)
    bl = b_left.reshape(1, _D)
    bi = b_inner.reshape(1, _D)
    br = b_roi.reshape(1, _D)

    def _prev(g):
        gm1 = jnp.maximum(g - 1, 0)
        return gm1 // _NRB, gm1 % _NRB

    out = pl.pallas_call(
        _roi_kernel,
        out_shape=jax.ShapeDtypeStruct((_B, _R, _D), jnp.float32),
        grid=(_NSTEPS + 1,),
        in_specs=[
            pl.BlockSpec(memory_space=pltpu.SMEM),
            pl.BlockSpec(memory_space=pltpu.SMEM),
            pl.BlockSpec((1, _T, _D),
                         lambda g: (jnp.minimum(g, _NSTEPS - 1) // _NRB, 0, 0)),
            pl.BlockSpec((7, _D, _D), lambda g: (0, 0, 0)),
            pl.BlockSpec((9, _D, _D), lambda g: (0, 0, 0)),
            pl.BlockSpec((2, _D, _D), lambda g: (0, 0, 0)),
            pl.BlockSpec((1, _D), lambda g: (0, 0)),
            pl.BlockSpec((1, _D), lambda g: (0, 0)),
            pl.BlockSpec((1, _D), lambda g: (0, 0)),
        ],
        out_specs=pl.BlockSpec((1, _RBLK, _D),
                               lambda g: (*_prev(g), 0)),
        scratch_shapes=[pltpu.VMEM((2 * _THALF, 1, _D), jnp.float32),
                        pltpu.VMEM((_T, 1, _D), jnp.float32)],
        compiler_params=pltpu.CompilerParams(
            dimension_semantics=("arbitrary",),
            vmem_limit_bytes=60 * 1024 * 1024,
        ),
        name="roi_relation_fused",
    )(lo_flat, w_flat, features, wl, wi, wr, bl, bi, br)
    return out


# speedup vs baseline: 1.1956x; 1.1956x over previous
import jax
import jax.numpy as jnp
from jax.experimental import pallas as pl
from jax.experimental.pallas import tpu as pltpu

_B, _T, _D, _R = 4, 2048, 512, 1024
_N_INNER, _N_B = 9, 2
_RATIO = 1.0 / 5.0
_NB = _N_B + _N_INNER + _N_B
_RBLK = 128
_NRB = _R // _RBLK
_NSTEPS = _B * _NRB
_THALF = _NB * _RBLK

_SELU_ALPHA = 1.6732632423543772
_SELU_SCALE = 1.0507009873554805


def _selu(x):
    return _SELU_SCALE * jnp.where(x > 0, x, _SELU_ALPHA * (jnp.exp(x) - 1.0))


def _roi_kernel(lo_ref,
                w_ref,
                feats_ref,
                wl_ref,
                wi_ref,
                wr_ref,
                bl_ref, bi_ref, br_ref,
                out_ref,
                tile_a,
                tile_b,
                f1_ref):
    g = pl.program_id(0)

    @pl.when(jnp.logical_and(g % _NRB == 0, g < _NSTEPS))
    def _():
        for c in range(_T // 256):
            f1_ref[pl.ds(c * 256, 256), 0, :] = feats_ref[0, pl.ds(c * 256, 256), :]

    def body(gt_ref, mt_ref):
        blk = jnp.minimum(g, _NSTEPS - 1)
        blk_base = blk * _RBLK * _NB
        for mi in range(_RBLK):
            for nb in range(_NB):
                idx = blk_base + mi * _NB + nb
                lo = lo_ref[idx]
                w = w_ref[idx]
                pair = f1_ref[pl.ds(lo, 2), 0, :]
                v = pair[0:1, :] + w * (pair[1:2, :] - pair[0:1, :])
                gt_ref[pl.ds(nb * _RBLK + mi, 1), 0, :] = v

        def xv(nb):
            return mt_ref[pl.ds(nb * _RBLK, _RBLK), 0, :]

        def dot(x, w):
            return jnp.dot(x, w, preferred_element_type=jnp.float32)

        acc_l = dot(xv(0), wl_ref[0])
        for j in range(1, 7):
            acc_l = acc_l + dot(xv(j), wl_ref[j])
        left = _selu(acc_l + bl_ref[0, :])

        acc_r = dot(xv(6), wl_ref[0])
        for j in range(1, 7):
            acc_r = acc_r + dot(xv(6 + j), wl_ref[j])
        right = _selu(acc_r + bl_ref[0, :])

        part1 = dot(right - left, wr_ref[0])

        acc_i = dot(xv(2), wi_ref[0])
        for j in range(1, 9):
            acc_i = acc_i + dot(xv(2 + j), wi_ref[j])
        inner = _selu(acc_i + bi_ref[0, :])

        out_ref[0, :, :] = _selu(part1 + dot(inner, wr_ref[1]) + br_ref[0, :])

    @pl.when(g % 2 == 0)
    def _():
        body(tile_a, tile_b)

    @pl.when(g % 2 == 1)
    def _():
        body(tile_b, tile_a)


def _bin_centers(a, b, n):
    i = (jnp.arange(n, dtype=a.dtype) + 0.5) / n
    return a[..., None] + i * (b - a)[..., None]


@jax.jit
def kernel(features, start_rois, end_rois, rois, rois_mask, rois_pos_emb,
           W_left, b_left, W_inner, b_inner, W_roi, b_roi):
    del start_rois, end_rois, rois_mask, rois_pos_emb

    s, e = rois[..., 0], rois[..., 1]
    ext = _RATIO * (e - s)
    pos = jnp.concatenate([
        _bin_centers(s - ext, s + ext, _N_B),
        _bin_centers(s, e, _N_INNER),
        _bin_centers(e - ext, e + ext, _N_B)], axis=-1)
    pos = jnp.clip(pos, 0.0, _T - 1)
    lo = jnp.minimum(pos.astype(jnp.int32), _T - 2)
    w = pos - lo.astype(jnp.float32)
    lo_flat = lo.reshape(-1)
    w_flat = w.reshape(-1)

    wl = W_left.reshape(_D, _D, 7).transpose(2, 1, 0)
    wi = W_inner.reshape(_D, _D, 9).transpose(2, 1, 0)
    wr = W_roi.T.reshape(2, _D, _D)
    bl = b_left.reshape(1, _D)
    bi = b_inner.reshape(1, _D)
    br = b_roi.reshape(1, _D)

    def _prev(g):
        gm1 = jnp.maximum(g - 1, 0)
        return gm1 // _NRB, gm1 % _NRB

    out = pl.pallas_call(
        _roi_kernel,
        out_shape=jax.ShapeDtypeStruct((_B, _R, _D), jnp.float32),
        grid=(_NSTEPS + 1,),
        in_specs=[
            pl.BlockSpec(memory_space=pltpu.SMEM),
            pl.BlockSpec(memory_space=pltpu.SMEM),
            pl.BlockSpec((1, _T, _D),
                         lambda g: (jnp.minimum(g, _NSTEPS - 1) // _NRB, 0, 0)),
            pl.BlockSpec((7, _D, _D), lambda g: (0, 0, 0)),
            pl.BlockSpec((9, _D, _D), lambda g: (0, 0, 0)),
            pl.BlockSpec((2, _D, _D), lambda g: (0, 0, 0)),
            pl.BlockSpec((1, _D), lambda g: (0, 0)),
            pl.BlockSpec((1, _D), lambda g: (0, 0)),
            pl.BlockSpec((1, _D), lambda g: (0, 0)),
        ],
        out_specs=pl.BlockSpec((1, _RBLK, _D),
                               lambda g: (*_prev(g), 0)),
        scratch_shapes=[pltpu.VMEM((_THALF, 1, _D), jnp.float32),
                        pltpu.VMEM((_THALF, 1, _D), jnp.float32),
                        pltpu.VMEM((_T, 1, _D), jnp.float32)],
        compiler_params=pltpu.CompilerParams(
            dimension_semantics=("arbitrary",),
            vmem_limit_bytes=60 * 1024 * 1024,
        ),
        name="roi_relation_fused",
    )(lo_flat, w_flat, features, wl, wi, wr, bl, bi, br)
    return out
